# Initial kernel scaffold; baseline (speedup 1.0000x reference)
import jax
import jax.numpy as jnp
from jax.experimental import pallas as pl


def kernel(gat_out, edge_index, W_in, b_in, W1, b1, W2, b2, clearance):
    raise NotImplementedError("write your pallas kernel here")


# speedup vs baseline: 5.7530x; 5.7530x over previous
import functools

import jax
import jax.numpy as jnp
from jax import lax
from jax.experimental import pallas as pl
from jax.experimental.pallas import tpu as pltpu
from jax.experimental.pallas import tpu_sc as plsc

N = 10000
E = 320000
D = 128
H2 = 256
STEPS = 12
DT = 6.0 / STEPS

NC = 2
NS = 16
NW = NC * NS
E_PER_TILE = E // NW
CHUNK = 80
NCHUNKS = E_PER_TILE // CHUNK
NPAD = 10240
ROWS_PER_TILE = NPAD // NS

_MESH = plsc.VectorSubcoreMesh(core_axis_name="c", subcore_axis_name="s")


@functools.partial(
    pl.kernel,
    out_type=jax.ShapeDtypeStruct((NC, NPAD, D), jnp.float32),
    mesh=_MESH,
    scratch_types=[
        pltpu.VMEM((NCHUNKS, CHUNK), jnp.int32),
        pltpu.VMEM((NCHUNKS, CHUNK), jnp.int32),
        pltpu.VMEM((CHUNK, D), jnp.float32),
        pltpu.VMEM_SHARED((NPAD, D), jnp.float32),
        pltpu.SemaphoreType.DMA,
    ],
)
def _sc_aggregate(h_hbm, src_hbm, dst_hbm, zero_hbm, out_hbm,
                  src_v, dst_v, rows_v, acc_sh, sem):
    c = lax.axis_index("c")
    s = lax.axis_index("s")
    wid = c * NS + s
    pltpu.sync_copy(src_hbm.at[wid], src_v)
    pltpu.sync_copy(dst_hbm.at[wid], dst_v)
    r0 = s * ROWS_PER_TILE
    pltpu.sync_copy(zero_hbm.at[pl.ds(r0, ROWS_PER_TILE)],
                    acc_sh.at[pl.ds(r0, ROWS_PER_TILE)])
    plsc.subcore_barrier()

    @pl.loop(0, NCHUNKS)
    def _chunk(g):
        pltpu.async_copy(h_hbm.at[src_v.at[g]], rows_v, sem).wait()
        pltpu.sync_copy(rows_v, acc_sh.at[dst_v.at[g]], add=True)

    plsc.subcore_barrier()
    pltpu.sync_copy(acc_sh.at[pl.ds(r0, ROWS_PER_TILE)],
                    out_hbm.at[c].at[pl.ds(r0, ROWS_PER_TILE)])


BLK = 1000


def _tc_in_body(x_ref, w_ref, b_ref, o_ref):
    o_ref[...] = jnp.tanh(
        jnp.dot(x_ref[...], w_ref[...], preferred_element_type=jnp.float32)
        + b_ref[...])


_tc_in = pl.pallas_call(
    _tc_in_body,
    grid=(N // BLK,),
    in_specs=[
        pl.BlockSpec((BLK, D), lambda i: (i, 0)),
        pl.BlockSpec((D, D), lambda i: (0, 0)),
        pl.BlockSpec((1, D), lambda i: (0, 0)),
    ],
    out_specs=pl.BlockSpec((BLK, D), lambda i: (i, 0)),
    out_shape=jax.ShapeDtypeStruct((N, D), jnp.float32),
)


def _tc_step_body(p0_ref, p1_ref, d0_ref, d1_ref, h_ref,
                  w1_ref, b1_ref, w2_ref, b2_ref, cl_ref, o_ref):
    deg = jnp.maximum(d0_ref[:, 0:1] + d1_ref[:, 0:1], 1.0)
    agg = (p0_ref[...] + p1_ref[...]) / deg
    z = jnp.dot(agg, w1_ref[...], preferred_element_type=jnp.float32) + b1_ref[...]
    z = 0.5 * z * (1.0 + lax.erf(z * (2.0 ** -0.5)))
    diff = jnp.tanh(
        jnp.dot(z, w2_ref[...], preferred_element_type=jnp.float32) + b2_ref[...])
    clr = jnp.maximum(cl_ref[0, 0], 0.0)
    o_ref[...] = h_ref[...] * (1.0 - clr * DT) + diff * DT


_tc_step = pl.pallas_call(
    _tc_step_body,
    grid=(N // BLK,),
    in_specs=[
        pl.BlockSpec((BLK, D), lambda i: (i, 0)),
        pl.BlockSpec((BLK, D), lambda i: (i, 0)),
        pl.BlockSpec((BLK, D), lambda i: (i, 0)),
        pl.BlockSpec((BLK, D), lambda i: (i, 0)),
        pl.BlockSpec((BLK, D), lambda i: (i, 0)),
        pl.BlockSpec((D, H2), lambda i: (0, 0)),
        pl.BlockSpec((1, H2), lambda i: (0, 0)),
        pl.BlockSpec((H2, D), lambda i: (0, 0)),
        pl.BlockSpec((1, D), lambda i: (0, 0)),
        pl.BlockSpec((1, 1), lambda i: (0, 0)),
    ],
    out_specs=pl.BlockSpec((BLK, D), lambda i: (i, 0)),
    out_shape=jax.ShapeDtypeStruct((N, D), jnp.float32),
)


def kernel(gat_out, edge_index, W_in, b_in, W1, b1, W2, b2, clearance):
    src = edge_index[0].astype(jnp.int32).reshape(NW, NCHUNKS, CHUNK)
    dst = edge_index[1].astype(jnp.int32).reshape(NW, NCHUNKS, CHUNK)
    zeros_nd = jnp.zeros((NPAD, D), jnp.float32)
    ones_nd = jnp.ones((N, D), jnp.float32)

    h0 = _tc_in(gat_out, W_in, b_in.reshape(1, D))
    degp = _sc_aggregate(ones_nd, src, dst, zeros_nd)
    d0 = degp[0]
    d1 = degp[1]
    b1r = b1.reshape(1, H2)
    b2r = b2.reshape(1, D)
    clr = clearance.reshape(1, 1)

    def step(_, h):
        p = _sc_aggregate(h, src, dst, zeros_nd)
        return _tc_step(p[0], p[1], d0, d1, h, W1, b1r, W2, b2r, clr)

    return lax.fori_loop(0, STEPS, step, h0)


# speedup vs baseline: 8.4049x; 1.4610x over previous
import functools

import jax
import jax.numpy as jnp
from jax import lax
from jax.experimental import pallas as pl
from jax.experimental.pallas import tpu as pltpu
from jax.experimental.pallas import tpu_sc as plsc

N = 10000
E = 320000
D = 128
H2 = 256
STEPS = 12
DT = 6.0 / STEPS

NC = 2
NS = 16
NW = NC * NS
E_PER_TILE = E // NW
CHUNK = 100
NCHUNKS = E_PER_TILE // CHUNK
NPAD = 10240
ROWS_PER_TILE = NPAD // NS

_MESH = plsc.VectorSubcoreMesh(core_axis_name="c", subcore_axis_name="s")


@functools.partial(
    pl.kernel,
    out_type=jax.ShapeDtypeStruct((NC, NPAD, D), jnp.float32),
    mesh=_MESH,
    scratch_types=[
        pltpu.VMEM((CHUNK,), jnp.int32),
        pltpu.VMEM((CHUNK,), jnp.int32),
        pltpu.VMEM((CHUNK,), jnp.int32),
        pltpu.VMEM((CHUNK,), jnp.int32),
        pltpu.VMEM((CHUNK, D), jnp.float32),
        pltpu.VMEM((CHUNK, D), jnp.float32),
        pltpu.VMEM_SHARED((NPAD, D), jnp.float32),
        pltpu.SemaphoreType.DMA,
        pltpu.SemaphoreType.DMA,
    ],
)
def _sc_aggregate(h_hbm, src_hbm, dst_hbm, zero_hbm, out_hbm,
                  src0, src1, dst0, dst1, rows0, rows1, acc_sh, isem, gsem):
    c = lax.axis_index("c")
    s = lax.axis_index("s")
    wid = c * NS + s
    my_src = src_hbm.at[wid]
    my_dst = dst_hbm.at[wid]
    pltpu.sync_copy(my_src.at[0], src0)
    pltpu.sync_copy(my_dst.at[0], dst0)
    pltpu.async_copy(h_hbm.at[src0], rows0, gsem)
    pltpu.async_copy(my_src.at[1], src1, isem)
    pltpu.async_copy(my_dst.at[1], dst1, isem)
    r0 = s * ROWS_PER_TILE
    pltpu.sync_copy(zero_hbm.at[pl.ds(r0, ROWS_PER_TILE)],
                    acc_sh.at[pl.ds(r0, ROWS_PER_TILE)])
    plsc.subcore_barrier()

    @pl.loop(0, NCHUNKS, step=2)
    def _outer(g0):
        for bufs in ((src0, dst0, rows0, src1, dst1, rows1),
                     (src1, dst1, rows1, src0, dst0, rows0)):
            sb, db, rb, so, do, ro = bufs
            g = g0 + (0 if sb is src0 else 1)

            @pl.when(g + 1 < NCHUNKS)
            def _():
                pltpu.make_async_copy(my_src.at[g + 1], so, isem).wait()
                pltpu.make_async_copy(my_dst.at[g + 1], do, isem).wait()
                pltpu.async_copy(h_hbm.at[so], ro, gsem)

            pltpu.make_async_copy(h_hbm.at[sb], rb, gsem).wait()
            pltpu.sync_copy(rb, acc_sh.at[db], add=True)

            @pl.when(g + 2 < NCHUNKS)
            def _():
                pltpu.async_copy(my_src.at[g + 2], sb, isem)
                pltpu.async_copy(my_dst.at[g + 2], db, isem)

    plsc.subcore_barrier()
    pltpu.sync_copy(acc_sh.at[pl.ds(r0, ROWS_PER_TILE)],
                    out_hbm.at[c].at[pl.ds(r0, ROWS_PER_TILE)])


BLK = 1000


def _tc_in_body(x_ref, w_ref, b_ref, o_ref):
    o_ref[...] = jnp.tanh(
        jnp.dot(x_ref[...], w_ref[...], preferred_element_type=jnp.float32)
        + b_ref[...])


_tc_in = pl.pallas_call(
    _tc_in_body,
    grid=(N // BLK,),
    in_specs=[
        pl.BlockSpec((BLK, D), lambda i: (i, 0)),
        pl.BlockSpec((D, D), lambda i: (0, 0)),
        pl.BlockSpec((1, D), lambda i: (0, 0)),
    ],
    out_specs=pl.BlockSpec((BLK, D), lambda i: (i, 0)),
    out_shape=jax.ShapeDtypeStruct((N, D), jnp.float32),
)


def _tc_step_body(p0_ref, p1_ref, d0_ref, d1_ref, h_ref,
                  w1_ref, b1_ref, w2_ref, b2_ref, cl_ref, o_ref):
    deg = jnp.maximum(d0_ref[:, 0:1] + d1_ref[:, 0:1], 1.0)
    agg = (p0_ref[...] + p1_ref[...]) / deg
    z = jnp.dot(agg, w1_ref[...], preferred_element_type=jnp.float32) + b1_ref[...]
    z = 0.5 * z * (1.0 + lax.erf(z * (2.0 ** -0.5)))
    diff = jnp.tanh(
        jnp.dot(z, w2_ref[...], preferred_element_type=jnp.float32) + b2_ref[...])
    clr = jnp.maximum(cl_ref[0, 0], 0.0)
    o_ref[...] = h_ref[...] * (1.0 - clr * DT) + diff * DT


_tc_step = pl.pallas_call(
    _tc_step_body,
    grid=(N // BLK,),
    in_specs=[
        pl.BlockSpec((BLK, D), lambda i: (i, 0)),
        pl.BlockSpec((BLK, D), lambda i: (i, 0)),
        pl.BlockSpec((BLK, D), lambda i: (i, 0)),
        pl.BlockSpec((BLK, D), lambda i: (i, 0)),
        pl.BlockSpec((BLK, D), lambda i: (i, 0)),
        pl.BlockSpec((D, H2), lambda i: (0, 0)),
        pl.BlockSpec((1, H2), lambda i: (0, 0)),
        pl.BlockSpec((H2, D), lambda i: (0, 0)),
        pl.BlockSpec((1, D), lambda i: (0, 0)),
        pl.BlockSpec((1, 1), lambda i: (0, 0)),
    ],
    out_specs=pl.BlockSpec((BLK, D), lambda i: (i, 0)),
    out_shape=jax.ShapeDtypeStruct((N, D), jnp.float32),
)


def kernel(gat_out, edge_index, W_in, b_in, W1, b1, W2, b2, clearance):
    src = edge_index[0].astype(jnp.int32).reshape(NW, NCHUNKS, CHUNK)
    dst = edge_index[1].astype(jnp.int32).reshape(NW, NCHUNKS, CHUNK)
    zeros_nd = jnp.zeros((NPAD, D), jnp.float32)
    ones_nd = jnp.ones((N, D), jnp.float32)

    h0 = _tc_in(gat_out, W_in, b_in.reshape(1, D))
    degp = _sc_aggregate(ones_nd, src, dst, zeros_nd)
    d0 = degp[0]
    d1 = degp[1]
    b1r = b1.reshape(1, H2)
    b2r = b2.reshape(1, D)
    clr = clearance.reshape(1, 1)

    def step(_, h):
        p = _sc_aggregate(h, src, dst, zeros_nd)
        return _tc_step(p[0], p[1], d0, d1, h, W1, b1r, W2, b2r, clr)

    return lax.fori_loop(0, STEPS, step, h0)


# speedup vs baseline: 9.0246x; 1.0737x over previous
import functools

import jax
import jax.numpy as jnp
from jax import lax
from jax.experimental import pallas as pl
from jax.experimental.pallas import tpu as pltpu
from jax.experimental.pallas import tpu_sc as plsc

N = 10000
E = 320000
D = 128
H2 = 256
STEPS = 12
DT = 6.0 / STEPS

NC = 2
NS = 16
NW = NC * NS
E_PER_TILE = E // NW
CHUNK = 125
NCHUNKS = E_PER_TILE // CHUNK
NPAD = 10240
ROWS_PER_TILE = NPAD // NS

_MESH = plsc.VectorSubcoreMesh(core_axis_name="c", subcore_axis_name="s")


@functools.partial(
    pl.kernel,
    out_type=jax.ShapeDtypeStruct((NC, NPAD, D), jnp.float32),
    mesh=_MESH,
    scratch_types=[
        pltpu.VMEM((CHUNK,), jnp.int32),
        pltpu.VMEM((CHUNK,), jnp.int32),
        pltpu.VMEM((CHUNK,), jnp.int32),
        pltpu.VMEM((CHUNK,), jnp.int32),
        pltpu.VMEM((CHUNK, D), jnp.float32),
        pltpu.VMEM((CHUNK, D), jnp.float32),
        pltpu.VMEM_SHARED((NPAD, D), jnp.float32),
        pltpu.SemaphoreType.DMA,
        pltpu.SemaphoreType.DMA,
    ],
)
def _sc_aggregate(h_hbm, src_hbm, dst_hbm, zero_hbm, out_hbm,
                  src0, src1, dst0, dst1, rows0, rows1, acc_sh, isem, gsem):
    c = lax.axis_index("c")
    s = lax.axis_index("s")
    wid = c * NS + s
    my_src = src_hbm.at[wid]
    my_dst = dst_hbm.at[wid]
    pltpu.sync_copy(my_src.at[0], src0)
    pltpu.sync_copy(my_dst.at[0], dst0)
    pltpu.async_copy(h_hbm.at[src0], rows0, gsem)
    pltpu.async_copy(my_src.at[1], src1, isem)
    pltpu.async_copy(my_dst.at[1], dst1, isem)
    r0 = s * ROWS_PER_TILE
    pltpu.sync_copy(zero_hbm.at[pl.ds(r0, ROWS_PER_TILE)],
                    acc_sh.at[pl.ds(r0, ROWS_PER_TILE)])
    plsc.subcore_barrier()

    @pl.loop(0, NCHUNKS, step=2)
    def _outer(g0):
        for bufs in ((src0, dst0, rows0, src1, dst1, rows1),
                     (src1, dst1, rows1, src0, dst0, rows0)):
            sb, db, rb, so, do, ro = bufs
            g = g0 + (0 if sb is src0 else 1)

            @pl.when(g + 1 < NCHUNKS)
            def _():
                pltpu.make_async_copy(my_src.at[g + 1], so, isem).wait()
                pltpu.make_async_copy(my_dst.at[g + 1], do, isem).wait()
                pltpu.async_copy(h_hbm.at[so], ro, gsem)

            pltpu.make_async_copy(h_hbm.at[sb], rb, gsem).wait()
            pltpu.sync_copy(rb, acc_sh.at[db], add=True)

            @pl.when(g + 2 < NCHUNKS)
            def _():
                pltpu.async_copy(my_src.at[g + 2], sb, isem)
                pltpu.async_copy(my_dst.at[g + 2], db, isem)

    plsc.subcore_barrier()
    pltpu.sync_copy(acc_sh.at[pl.ds(r0, ROWS_PER_TILE)],
                    out_hbm.at[c].at[pl.ds(r0, ROWS_PER_TILE)])


BLK = 1000


def _tc_in_body(x_ref, w_ref, b_ref, o_ref):
    o_ref[...] = jnp.tanh(
        jnp.dot(x_ref[...], w_ref[...], preferred_element_type=jnp.float32)
        + b_ref[...])


_tc_in = pl.pallas_call(
    _tc_in_body,
    grid=(N // BLK,),
    in_specs=[
        pl.BlockSpec((BLK, D), lambda i: (i, 0)),
        pl.BlockSpec((D, D), lambda i: (0, 0)),
        pl.BlockSpec((1, D), lambda i: (0, 0)),
    ],
    out_specs=pl.BlockSpec((BLK, D), lambda i: (i, 0)),
    out_shape=jax.ShapeDtypeStruct((N, D), jnp.float32),
)


def _tc_step_body(p0_ref, p1_ref, d0_ref, d1_ref, h_ref,
                  w1_ref, b1_ref, w2_ref, b2_ref, cl_ref, o_ref):
    deg = jnp.maximum(d0_ref[:, 0:1] + d1_ref[:, 0:1], 1.0)
    agg = (p0_ref[...] + p1_ref[...]) / deg
    z = jnp.dot(agg, w1_ref[...], preferred_element_type=jnp.float32) + b1_ref[...]
    z = 0.5 * z * (1.0 + lax.erf(z * (2.0 ** -0.5)))
    diff = jnp.tanh(
        jnp.dot(z, w2_ref[...], preferred_element_type=jnp.float32) + b2_ref[...])
    clr = jnp.maximum(cl_ref[0, 0], 0.0)
    o_ref[...] = h_ref[...] * (1.0 - clr * DT) + diff * DT


_tc_step = pl.pallas_call(
    _tc_step_body,
    grid=(N // BLK,),
    in_specs=[
        pl.BlockSpec((BLK, D), lambda i: (i, 0)),
        pl.BlockSpec((BLK, D), lambda i: (i, 0)),
        pl.BlockSpec((BLK, D), lambda i: (i, 0)),
        pl.BlockSpec((BLK, D), lambda i: (i, 0)),
        pl.BlockSpec((BLK, D), lambda i: (i, 0)),
        pl.BlockSpec((D, H2), lambda i: (0, 0)),
        pl.BlockSpec((1, H2), lambda i: (0, 0)),
        pl.BlockSpec((H2, D), lambda i: (0, 0)),
        pl.BlockSpec((1, D), lambda i: (0, 0)),
        pl.BlockSpec((1, 1), lambda i: (0, 0)),
    ],
    out_specs=pl.BlockSpec((BLK, D), lambda i: (i, 0)),
    out_shape=jax.ShapeDtypeStruct((N, D), jnp.float32),
)


def kernel(gat_out, edge_index, W_in, b_in, W1, b1, W2, b2, clearance):
    src = edge_index[0].astype(jnp.int32).reshape(NW, NCHUNKS, CHUNK)
    dst = edge_index[1].astype(jnp.int32).reshape(NW, NCHUNKS, CHUNK)
    zeros_nd = jnp.zeros((NPAD, D), jnp.float32)
    ones_nd = jnp.ones((N, D), jnp.float32)

    h0 = _tc_in(gat_out, W_in, b_in.reshape(1, D))
    degp = _sc_aggregate(ones_nd, src, dst, zeros_nd)
    d0 = degp[0]
    d1 = degp[1]
    b1r = b1.reshape(1, H2)
    b2r = b2.reshape(1, D)
    clr = clearance.reshape(1, 1)

    def step(_, h):
        p = _sc_aggregate(h, src, dst, zeros_nd)
        return _tc_step(p[0], p[1], d0, d1, h, W1, b1r, W2, b2r, clr)

    return lax.fori_loop(0, STEPS, step, h0)


# speedup vs baseline: 10.1585x; 1.1256x over previous
import functools

import jax
import jax.numpy as jnp
from jax import lax
from jax.experimental import pallas as pl
from jax.experimental.pallas import tpu as pltpu
from jax.experimental.pallas import tpu_sc as plsc

N = 10000
E = 320000
D = 128
H2 = 256
STEPS = 12
DT = 6.0 / STEPS

NC = 2
NS = 16
NW = NC * NS
E_PER_TILE = E // NW
CHUNK = 125
NCHUNKS = E_PER_TILE // CHUNK
NPAD = 10240
ROWS_PER_TILE = NPAD // NS

_MESH = plsc.VectorSubcoreMesh(core_axis_name="c", subcore_axis_name="s")


@functools.partial(
    pl.kernel,
    out_type=jax.ShapeDtypeStruct((NC, NPAD, D), jnp.float32),
    mesh=_MESH,
    scratch_types=[
        [pltpu.VMEM((CHUNK,), jnp.int32)] * 4,
        [pltpu.VMEM((CHUNK,), jnp.int32)] * 4,
        [pltpu.VMEM((CHUNK, D), jnp.float32)] * 2,
        pltpu.VMEM_SHARED((NPAD, D), jnp.float32),
        pltpu.SemaphoreType.DMA,
        pltpu.SemaphoreType.DMA,
        [pltpu.SemaphoreType.DMA] * 2,
    ],
)
def _sc_aggregate(h_hbm, src_hbm, dst_hbm, zero_hbm, out_hbm,
                  S, Dx, R, acc_sh, isem, gsem, SS):
    c = lax.axis_index("c")
    s = lax.axis_index("s")
    wid = c * NS + s
    my_src = src_hbm.at[wid]
    my_dst = dst_hbm.at[wid]
    pltpu.sync_copy(my_src.at[0], S[0])
    pltpu.sync_copy(my_dst.at[0], Dx[0])
    pltpu.async_copy(h_hbm.at[S[0]], R[0], gsem)
    for k in (1, 2):
        pltpu.async_copy(my_src.at[k], S[k], isem)
        pltpu.async_copy(my_dst.at[k], Dx[k], isem)
    r0 = s * ROWS_PER_TILE
    pltpu.sync_copy(zero_hbm.at[pl.ds(r0, ROWS_PER_TILE)],
                    acc_sh.at[pl.ds(r0, ROWS_PER_TILE)])
    plsc.subcore_barrier()

    @pl.loop(0, NCHUNKS, step=4)
    def _outer(g0):
        for j in range(4):
            g = g0 + j
            sb, db, rb = S[j], Dx[j], R[j % 2]
            so, do_, ro = S[(j + 1) % 4], Dx[(j + 1) % 4], R[(j + 1) % 2]
            sp, dp = S[(j + 3) % 4], Dx[(j + 3) % 4]

            @pl.when(g + 1 < NCHUNKS)
            def _():
                @pl.when(g >= 1)
                def _():
                    pltpu.make_async_copy(ro, acc_sh.at[do_],
                                          SS[(j + 1) % 2]).wait()
                pltpu.make_async_copy(my_src.at[g + 1], so, isem).wait()
                pltpu.make_async_copy(my_dst.at[g + 1], do_, isem).wait()
                pltpu.async_copy(h_hbm.at[so], ro, gsem)

                @pl.when(g + 3 < NCHUNKS)
                def _():
                    pltpu.async_copy(my_src.at[g + 3], sp, isem)
                    pltpu.async_copy(my_dst.at[g + 3], dp, isem)

            pltpu.make_async_copy(h_hbm.at[sb], rb, gsem).wait()
            pltpu.async_copy(rb, acc_sh.at[db], SS[j % 2], add=True)

    pltpu.make_async_copy(R[0], acc_sh.at[Dx[2]], SS[0]).wait()
    pltpu.make_async_copy(R[1], acc_sh.at[Dx[3]], SS[1]).wait()
    plsc.subcore_barrier()
    pltpu.sync_copy(acc_sh.at[pl.ds(r0, ROWS_PER_TILE)],
                    out_hbm.at[c].at[pl.ds(r0, ROWS_PER_TILE)])


BLK = 1000


def _tc_in_body(x_ref, w_ref, b_ref, o_ref):
    o_ref[...] = jnp.tanh(
        jnp.dot(x_ref[...], w_ref[...], preferred_element_type=jnp.float32)
        + b_ref[...])


_tc_in = pl.pallas_call(
    _tc_in_body,
    grid=(N // BLK,),
    in_specs=[
        pl.BlockSpec((BLK, D), lambda i: (i, 0)),
        pl.BlockSpec((D, D), lambda i: (0, 0)),
        pl.BlockSpec((1, D), lambda i: (0, 0)),
    ],
    out_specs=pl.BlockSpec((BLK, D), lambda i: (i, 0)),
    out_shape=jax.ShapeDtypeStruct((N, D), jnp.float32),
)


def _tc_step_body(p0_ref, p1_ref, d0_ref, d1_ref, h_ref,
                  w1_ref, b1_ref, w2_ref, b2_ref, cl_ref, o_ref):
    deg = jnp.maximum(d0_ref[:, 0:1] + d1_ref[:, 0:1], 1.0)
    agg = (p0_ref[...] + p1_ref[...]) / deg
    z = jnp.dot(agg, w1_ref[...], preferred_element_type=jnp.float32) + b1_ref[...]
    z = 0.5 * z * (1.0 + lax.erf(z * (2.0 ** -0.5)))
    diff = jnp.tanh(
        jnp.dot(z, w2_ref[...], preferred_element_type=jnp.float32) + b2_ref[...])
    clr = jnp.maximum(cl_ref[0, 0], 0.0)
    o_ref[...] = h_ref[...] * (1.0 - clr * DT) + diff * DT


_tc_step = pl.pallas_call(
    _tc_step_body,
    grid=(N // BLK,),
    in_specs=[
        pl.BlockSpec((BLK, D), lambda i: (i, 0)),
        pl.BlockSpec((BLK, D), lambda i: (i, 0)),
        pl.BlockSpec((BLK, D), lambda i: (i, 0)),
        pl.BlockSpec((BLK, D), lambda i: (i, 0)),
        pl.BlockSpec((BLK, D), lambda i: (i, 0)),
        pl.BlockSpec((D, H2), lambda i: (0, 0)),
        pl.BlockSpec((1, H2), lambda i: (0, 0)),
        pl.BlockSpec((H2, D), lambda i: (0, 0)),
        pl.BlockSpec((1, D), lambda i: (0, 0)),
        pl.BlockSpec((1, 1), lambda i: (0, 0)),
    ],
    out_specs=pl.BlockSpec((BLK, D), lambda i: (i, 0)),
    out_shape=jax.ShapeDtypeStruct((N, D), jnp.float32),
)


def kernel(gat_out, edge_index, W_in, b_in, W1, b1, W2, b2, clearance):
    src = edge_index[0].astype(jnp.int32).reshape(NW, NCHUNKS, CHUNK)
    dst = edge_index[1].astype(jnp.int32).reshape(NW, NCHUNKS, CHUNK)
    zeros_nd = jnp.zeros((NPAD, D), jnp.float32)
    ones_nd = jnp.ones((N, D), jnp.float32)

    h0 = _tc_in(gat_out, W_in, b_in.reshape(1, D))
    degp = _sc_aggregate(ones_nd, src, dst, zeros_nd)
    d0 = degp[0]
    d1 = degp[1]
    b1r = b1.reshape(1, H2)
    b2r = b2.reshape(1, D)
    clr = clearance.reshape(1, 1)

    def step(_, h):
        p = _sc_aggregate(h, src, dst, zeros_nd)
        return _tc_step(p[0], p[1], d0, d1, h, W1, b1r, W2, b2r, clr)

    return lax.fori_loop(0, STEPS, step, h0)


# speedup vs baseline: 10.3227x; 1.0162x over previous
import functools

import jax
import jax.numpy as jnp
from jax import lax
from jax.experimental import pallas as pl
from jax.experimental.pallas import tpu as pltpu
from jax.experimental.pallas import tpu_sc as plsc

N = 10000
E = 320000
D = 128
H2 = 256
STEPS = 12
DT = 6.0 / STEPS

NC = 2
NS = 16
NW = NC * NS
E_PER_TILE = E // NW
CHUNK = 125
NCHUNKS = E_PER_TILE // CHUNK
NPAD = 10240
ROWS_PER_TILE = NPAD // NS

_MESH = plsc.VectorSubcoreMesh(core_axis_name="c", subcore_axis_name="s")


@functools.partial(
    pl.kernel,
    out_type=jax.ShapeDtypeStruct((NC, NPAD, D), jnp.float32),
    mesh=_MESH,
    scratch_types=[
        [pltpu.VMEM((CHUNK,), jnp.int32)] * 4,
        [pltpu.VMEM((CHUNK,), jnp.int32)] * 4,
        [pltpu.VMEM((CHUNK, D), jnp.float32)] * 2,
        pltpu.VMEM_SHARED((NPAD, D), jnp.float32),
        pltpu.SemaphoreType.DMA,
        pltpu.SemaphoreType.DMA,
        [pltpu.SemaphoreType.DMA] * 2,
    ],
)
def _sc_aggregate(h_hbm, src_hbm, dst_hbm, zero_hbm, out_hbm,
                  S, Dx, R, acc_sh, isem, gsem, SS):
    c = lax.axis_index("c")
    s = lax.axis_index("s")
    wid = c * NS + s
    my_src = src_hbm.at[wid]
    my_dst = dst_hbm.at[wid]
    pltpu.sync_copy(my_src.at[0], S[0])
    pltpu.sync_copy(my_dst.at[0], Dx[0])
    pltpu.async_copy(h_hbm.at[S[0]], R[0], gsem)
    for k in (1, 2):
        pltpu.async_copy(my_src.at[k], S[k], isem)
        pltpu.async_copy(my_dst.at[k], Dx[k], isem)
    r0 = s * ROWS_PER_TILE
    pltpu.sync_copy(zero_hbm.at[pl.ds(r0, ROWS_PER_TILE)],
                    acc_sh.at[pl.ds(r0, ROWS_PER_TILE)])
    plsc.subcore_barrier()

    @pl.loop(0, NCHUNKS, step=4)
    def _outer(g0):
        for j in range(4):
            g = g0 + j
            sb, db, rb = S[j], Dx[j], R[j % 2]
            so, do_, ro = S[(j + 1) % 4], Dx[(j + 1) % 4], R[(j + 1) % 2]
            sp, dp = S[(j + 3) % 4], Dx[(j + 3) % 4]

            @pl.when(g + 1 < NCHUNKS)
            def _():
                @pl.when(g >= 1)
                def _():
                    pltpu.make_async_copy(ro, acc_sh.at[do_],
                                          SS[(j + 1) % 2]).wait()
                pltpu.make_async_copy(my_src.at[g + 1], so, isem).wait()
                pltpu.make_async_copy(my_dst.at[g + 1], do_, isem).wait()
                pltpu.async_copy(h_hbm.at[so], ro, gsem)

                @pl.when(g + 3 < NCHUNKS)
                def _():
                    pltpu.async_copy(my_src.at[g + 3], sp, isem)
                    pltpu.async_copy(my_dst.at[g + 3], dp, isem)

            pltpu.make_async_copy(h_hbm.at[sb], rb, gsem).wait()
            pltpu.async_copy(rb, acc_sh.at[db], SS[j % 2], add=True)

    pltpu.make_async_copy(R[0], acc_sh.at[Dx[2]], SS[0]).wait()
    pltpu.make_async_copy(R[1], acc_sh.at[Dx[3]], SS[1]).wait()
    plsc.subcore_barrier()
    pltpu.sync_copy(acc_sh.at[pl.ds(r0, ROWS_PER_TILE)],
                    out_hbm.at[c].at[pl.ds(r0, ROWS_PER_TILE)])


BLK = 2000


def _tc_in_body(x_ref, w_ref, b_ref, o_ref):
    o_ref[...] = jnp.tanh(
        jnp.dot(x_ref[...], w_ref[...], preferred_element_type=jnp.float32)
        + b_ref[...])


_tc_in = pl.pallas_call(
    _tc_in_body,
    grid=(N // BLK,),
    in_specs=[
        pl.BlockSpec((BLK, D), lambda i: (i, 0)),
        pl.BlockSpec((D, D), lambda i: (0, 0)),
        pl.BlockSpec((1, D), lambda i: (0, 0)),
    ],
    out_specs=pl.BlockSpec((BLK, D), lambda i: (i, 0)),
    out_shape=jax.ShapeDtypeStruct((N, D), jnp.float32),
)


def _tc_step_body(p0_ref, p1_ref, d0_ref, d1_ref, h_ref,
                  w1_ref, b1_ref, w2_ref, b2_ref, cl_ref, o_ref):
    deg = jnp.maximum(d0_ref[:, 0:1] + d1_ref[:, 0:1], 1.0)
    agg = (p0_ref[...] + p1_ref[...]) / deg
    z = jnp.dot(agg, w1_ref[...], preferred_element_type=jnp.float32) + b1_ref[...]
    z = 0.5 * z * (1.0 + lax.erf(z * (2.0 ** -0.5)))
    diff = jnp.tanh(
        jnp.dot(z, w2_ref[...], preferred_element_type=jnp.float32) + b2_ref[...])
    clr = jnp.maximum(cl_ref[0, 0], 0.0)
    o_ref[...] = h_ref[...] * (1.0 - clr * DT) + diff * DT


_tc_step = pl.pallas_call(
    _tc_step_body,
    grid=(N // BLK,),
    in_specs=[
        pl.BlockSpec((BLK, D), lambda i: (i, 0)),
        pl.BlockSpec((BLK, D), lambda i: (i, 0)),
        pl.BlockSpec((BLK, D), lambda i: (i, 0)),
        pl.BlockSpec((BLK, D), lambda i: (i, 0)),
        pl.BlockSpec((BLK, D), lambda i: (i, 0)),
        pl.BlockSpec((D, H2), lambda i: (0, 0)),
        pl.BlockSpec((1, H2), lambda i: (0, 0)),
        pl.BlockSpec((H2, D), lambda i: (0, 0)),
        pl.BlockSpec((1, D), lambda i: (0, 0)),
        pl.BlockSpec((1, 1), lambda i: (0, 0)),
    ],
    out_specs=pl.BlockSpec((BLK, D), lambda i: (i, 0)),
    out_shape=jax.ShapeDtypeStruct((N, D), jnp.float32),
)


def kernel(gat_out, edge_index, W_in, b_in, W1, b1, W2, b2, clearance):
    src = edge_index[0].astype(jnp.int32).reshape(NW, NCHUNKS, CHUNK)
    dst = edge_index[1].astype(jnp.int32).reshape(NW, NCHUNKS, CHUNK)
    zeros_nd = jnp.zeros((NPAD, D), jnp.float32)
    ones_nd = jnp.ones((N, D), jnp.float32)

    h0 = _tc_in(gat_out, W_in, b_in.reshape(1, D))
    degp = _sc_aggregate(ones_nd, src, dst, zeros_nd)
    d0 = degp[0]
    d1 = degp[1]
    b1r = b1.reshape(1, H2)
    b2r = b2.reshape(1, D)
    clr = clearance.reshape(1, 1)

    def step(_, h):
        p = _sc_aggregate(h, src, dst, zeros_nd)
        return _tc_step(p[0], p[1], d0, d1, h, W1, b1r, W2, b2r, clr)

    return lax.fori_loop(0, STEPS, step, h0)
